# baseline (speedup 1.0000x reference)
import jax
import jax.numpy as jnp
from jax.experimental import pallas as pl
from jax.experimental.pallas import tpu as pltpu

_EPS = 1e-12
_G = 8


def _part_vlad(x, wt):
    ssq = jnp.sum(x * x, axis=1, keepdims=True)
    rinv = jax.lax.rsqrt(jnp.maximum(ssq, _EPS * _EPS))
    logits = jnp.dot(x, wt, preferred_element_type=jnp.float32) * rinv
    m = jnp.max(logits, axis=1, keepdims=True)
    e = jnp.exp(logits - m)
    a = e / jnp.sum(e, axis=1, keepdims=True)
    a2 = a * rinv
    vlad = jax.lax.dot_general(
        a2, x, (((0,), (0,)), ((), ())), preferred_element_type=jnp.float32
    )
    asum = jnp.sum(a, axis=0, keepdims=True)
    return vlad, asum


def _finish(vlad, asum, c):
    vlad = vlad - asum.T * c
    rn = jnp.sqrt(jnp.sum(vlad * vlad, axis=1, keepdims=True))
    vlad = vlad / jnp.maximum(rn, _EPS)
    g = jnp.sqrt(jnp.sum(vlad * vlad, keepdims=True))
    return vlad / jnp.maximum(g, _EPS)


def _netvlad_block(x1_ref, x2_ref, x3_ref, x4_ref, wt_ref, c_ref, o_ref):
    wt = wt_ref[...]
    c = c_ref[...]
    for g in range(_G):
        v1, s1 = _part_vlad(x1_ref[g], wt)
        v2, s2 = _part_vlad(x2_ref[g], wt)
        v3, s3 = _part_vlad(x3_ref[g], wt)
        v4, s4 = _part_vlad(x4_ref[g], wt)
        o_ref[g] = _finish((v1 + v2) + (v3 + v4), (s1 + s2) + (s3 + s4), c)


@jax.jit
def kernel(x, conv_w, centroids):
    B, D, H, W = x.shape
    K = centroids.shape[0]
    N = H * W

    xt = jnp.transpose(x, (0, 2, 3, 1)).reshape(B, N, D)
    out = pl.pallas_call(
        _netvlad_block,
        grid=(B // _G,),
        in_specs=[
            pl.BlockSpec((_G, 320, D), lambda i: (i, 0, 0)),
            pl.BlockSpec((_G, 320, D), lambda i: (i, 1, 0)),
            pl.BlockSpec((_G, 320, D), lambda i: (i, 2, 0)),
            pl.BlockSpec((_G, 240, D), lambda i: (i, 4, 0)),
            pl.BlockSpec((D, K), lambda i: (0, 0)),
            pl.BlockSpec((K, D), lambda i: (0, 0)),
        ],
        out_specs=pl.BlockSpec((_G, K, D), lambda i: (i, 0, 0)),
        out_shape=jax.ShapeDtypeStruct((B, K, D), jnp.float32),
        compiler_params=pltpu.CompilerParams(
            dimension_semantics=("parallel",),
            vmem_limit_bytes=56 * 1024 * 1024,
        ),
    )(xt, xt, xt, xt, conv_w.T, centroids)
    return out.reshape(B, K * D)


# speedup vs baseline: 1.3052x; 1.3052x over previous
import jax
import jax.numpy as jnp
from jax.experimental import pallas as pl
from jax.experimental.pallas import tpu as pltpu

_EPS = 1e-12
_G = 8


def _part_vlad(x, wt):
    ssq = jnp.sum(x * x, axis=1, keepdims=True)
    rinv = jax.lax.rsqrt(jnp.maximum(ssq, _EPS * _EPS))
    logits = jnp.dot(x, wt, preferred_element_type=jnp.float32) * rinv
    m = jnp.max(logits, axis=1, keepdims=True)
    e = jnp.exp(logits - m)
    a = e / jnp.sum(e, axis=1, keepdims=True)
    a2 = a * rinv
    vlad = jax.lax.dot_general(
        a2, x, (((0,), (0,)), ((), ())), preferred_element_type=jnp.float32
    )
    asum = jnp.sum(a, axis=0, keepdims=True)
    return vlad, asum


def _finish(vlad, asum, c):
    vlad = vlad - asum.T * c
    rn = jnp.sqrt(jnp.sum(vlad * vlad, axis=1, keepdims=True))
    vlad = vlad / jnp.maximum(rn, _EPS)
    g = jnp.sqrt(jnp.sum(vlad * vlad, keepdims=True))
    return vlad / jnp.maximum(g, _EPS)


def _netvlad_block(x1_ref, x2_ref, wt_ref, c_ref, o_ref):
    wt = wt_ref[...]
    c = c_ref[...]
    for g in range(_G):
        v1, s1 = _part_vlad(x1_ref[g], wt)
        v2, s2 = _part_vlad(x2_ref[g], wt)
        o_ref[g] = _finish(v1 + v2, s1 + s2, c)


@jax.jit
def kernel(x, conv_w, centroids):
    B, D, H, W = x.shape
    K = centroids.shape[0]
    N = H * W
    Nh = N // 2
    xt = jnp.transpose(x, (0, 2, 3, 1)).reshape(B, N, D)
    out = pl.pallas_call(
        _netvlad_block,
        grid=(B // _G,),
        in_specs=[
            pl.BlockSpec((_G, Nh, D), lambda i: (i, 0, 0)),
            pl.BlockSpec((_G, Nh, D), lambda i: (i, 1, 0)),
            pl.BlockSpec((D, K), lambda i: (0, 0)),
            pl.BlockSpec((K, D), lambda i: (0, 0)),
        ],
        out_specs=pl.BlockSpec((_G, K, D), lambda i: (i, 0, 0)),
        out_shape=jax.ShapeDtypeStruct((B, K, D), jnp.float32),
        compiler_params=pltpu.CompilerParams(
            dimension_semantics=("parallel",),
            vmem_limit_bytes=56 * 1024 * 1024,
        ),
    )(xt, xt, conv_w.T, centroids)
    return out.reshape(B, K * D)
